# baseline (speedup 1.0000x reference)
import jax
import jax.numpy as jnp
from jax.experimental import pallas as pl
from jax.experimental.pallas import tpu as pltpu

_LOG2E = 1.4426950408889634


def _fused_attn_kernel(adj_ref, x_ref, hv_ref, out_ref, s_ref):
    @pl.when(pl.program_id(0) == 0)
    def _prologue():
        s = jnp.dot(x_ref[...], hv_ref[...],
                    preferred_element_type=jnp.float32)
        s_ref[...] = (s * _LOG2E).T

    s2 = s_ref[...]
    a = adj_ref[...]
    e = jnp.where(a != 0.0, jnp.exp2(a * s2), 0.0)
    denom = jnp.sum(e, axis=1, keepdims=True)
    acc = jnp.dot(e, x_ref[...],
                  preferred_element_type=jnp.float32)
    out_ref[...] = acc / denom


def kernel(inputs, adj, H_v):
    n, d = inputs.shape
    bm = 256
    grid = (n // bm,)
    return pl.pallas_call(
        _fused_attn_kernel,
        grid=grid,
        in_specs=[
            pl.BlockSpec((bm, n), lambda i: (i, 0)),
            pl.BlockSpec((n, d), lambda i: (0, 0)),
            pl.BlockSpec((d, 1), lambda i: (0, 0)),
        ],
        out_specs=pl.BlockSpec((bm, d), lambda i: (i, 0)),
        out_shape=jax.ShapeDtypeStruct((n, d), jnp.float32),
        scratch_shapes=[
            pltpu.VMEM((1, n), jnp.float32),
        ],
    )(adj, inputs, H_v)


# speedup vs baseline: 1.1305x; 1.1305x over previous
import jax
import jax.numpy as jnp
from jax.experimental import pallas as pl
from jax.experimental.pallas import tpu as pltpu

_LOG2E = 1.4426950408889634


def _fused_attn_kernel(adj_ref, x_ref, hv_ref, out_ref, s_ref):
    @pl.when(pl.program_id(0) == 0)
    def _prologue():
        s = jnp.dot(x_ref[...], hv_ref[...],
                    preferred_element_type=jnp.float32)
        s_ref[...] = (s * _LOG2E).T

    s2 = s_ref[...]
    a = adj_ref[...]
    e = jnp.where(a != 0.0, jnp.exp2(a * s2), 0.0)
    denom = jnp.sum(e, axis=1, keepdims=True)
    acc = jnp.dot(e, x_ref[...],
                  preferred_element_type=jnp.float32)
    out_ref[...] = acc / denom


def kernel(inputs, adj, H_v):
    n, d = inputs.shape
    bm = 512
    grid = (n // bm,)
    return pl.pallas_call(
        _fused_attn_kernel,
        grid=grid,
        in_specs=[
            pl.BlockSpec((bm, n), lambda i: (i, 0)),
            pl.BlockSpec((n, d), lambda i: (0, 0)),
            pl.BlockSpec((d, 1), lambda i: (0, 0)),
        ],
        out_specs=pl.BlockSpec((bm, d), lambda i: (i, 0)),
        out_shape=jax.ShapeDtypeStruct((n, d), jnp.float32),
        scratch_shapes=[
            pltpu.VMEM((1, n), jnp.float32),
        ],
    )(adj, inputs, H_v)
